# Initial kernel scaffold; baseline (speedup 1.0000x reference)
import jax
import jax.numpy as jnp
from jax.experimental import pallas as pl


def kernel(x, di):
    raise NotImplementedError("write your pallas kernel here")


# speedup vs baseline: 1.1111x; 1.1111x over previous
import functools

import jax
import jax.numpy as jnp
from jax import lax
from jax.experimental import pallas as pl
from jax.experimental.pallas import tpu as pltpu
from jax.experimental.pallas import tpu_sc as plsc

_INFO = plsc.get_sparse_core_info()
_NC = _INFO.num_cores
_NS = _INFO.num_subcores
_NW = _NC * _NS

_N = 16384 * 100
_D = 45
_DP = 48
_PER_W = _N // _NW
_WIN = 2048
_STEPS = _PER_W // _WIN


def _gather_body(x_hbm, di_hbm, out_hbm, idx_v, rows_v, sem):
    wid = lax.axis_index("s") * _NC + lax.axis_index("c")
    wbase = wid * _PER_W

    def step(i, _):
        base = wbase + i * _WIN
        pltpu.sync_copy(x_hbm.at[pl.ds(base, _WIN)], idx_v)
        pltpu.async_copy(di_hbm.at[idx_v], rows_v, sem).wait()
        pltpu.sync_copy(rows_v, out_hbm.at[pl.ds(base, _WIN)])
        return _

    lax.fori_loop(0, _STEPS, step, 0)


@jax.jit
def kernel(x, di):
    xf = x.reshape(_N)
    dip = jnp.pad(di, ((0, 0), (0, _DP - _D)))
    mesh = plsc.VectorSubcoreMesh(core_axis_name="c", subcore_axis_name="s")
    out = pl.kernel(
        _gather_body,
        mesh=mesh,
        out_type=jax.ShapeDtypeStruct((_N, _DP), jnp.float32),
        scratch_types=[
            pltpu.VMEM((_WIN,), jnp.int32),
            pltpu.VMEM((_WIN, _DP), jnp.float32),
            pltpu.SemaphoreType.DMA,
        ],
        compiler_params=pltpu.CompilerParams(use_tc_tiling_on_sc=False),
    )(xf, dip)
    return out[:, :_D].reshape(x.shape[0], x.shape[1], _D)
